# baseline (speedup 1.0000x reference)
import functools

import jax
import jax.numpy as jnp
from jax import lax
from jax.experimental import pallas as pl
from jax.experimental.pallas import tpu as pltpu
from jax.experimental.pallas import tpu_sc as plsc

BATCH = 4096
SEQ = 200
DIM = 64
NUM_CLASSES = 10

NC = 2
NS = 16
NW = NC * NS
ROWS_PER_W = BATCH // NW
HALF = SEQ // 2
LANES = 16
UNROLL = 8


def _pooled_mean(x2d, emb):
    mesh = plsc.VectorSubcoreMesh(core_axis_name="c", subcore_axis_name="s")

    @functools.partial(
        pl.kernel,
        out_type=jax.ShapeDtypeStruct((BATCH, DIM), jnp.float32),
        mesh=mesh,
        scratch_types=[
            pltpu.VMEM((2 * ROWS_PER_W, HALF), jnp.int32),
            pltpu.VMEM((SEQ, DIM), jnp.float32),
            pltpu.VMEM((ROWS_PER_W, DIM), jnp.float32),
            pltpu.SemaphoreType.DMA,
        ],
        compiler_params=pltpu.CompilerParams(use_tc_tiling_on_sc=False),
    )
    def k(x_hbm, emb_hbm, out_hbm, idx_v, rows_v, out_v, sem):
        cid = lax.axis_index("c")
        sid = lax.axis_index("s")
        wid = sid * NC + cid

        pltpu.sync_copy(
            x_hbm.at[pl.ds(wid * (2 * ROWS_PER_W), 2 * ROWS_PER_W)], idx_v
        )

        def seg_body(s, carry):
            h0 = pltpu.async_copy(
                emb_hbm.at[idx_v.at[2 * s]], rows_v.at[pl.ds(0, HALF)], sem
            )
            h1 = pltpu.async_copy(
                emb_hbm.at[idx_v.at[2 * s + 1]], rows_v.at[pl.ds(HALF, HALF)], sem
            )
            h0.wait()
            h1.wait()

            def red(i, acc):
                accs = list(acc)
                for r in range(UNROLL):
                    row = i * UNROLL + r
                    for q in range(DIM // LANES):
                        accs[q] = accs[q] + rows_v[row, pl.ds(q * LANES, LANES)]
                return tuple(accs)

            zero = jnp.zeros((LANES,), jnp.float32)
            acc = lax.fori_loop(0, SEQ // UNROLL, red, (zero,) * (DIM // LANES))
            for q in range(DIM // LANES):
                out_v[s, pl.ds(q * LANES, LANES)] = acc[q] * (1.0 / SEQ)
            return carry

        lax.fori_loop(0, ROWS_PER_W, seg_body, 0)
        pltpu.sync_copy(out_v, out_hbm.at[pl.ds(wid * ROWS_PER_W, ROWS_PER_W)])

    return k(x2d, emb)


def _mlp(pooled, W1, b1, W2, b2):
    def mk(p_ref, w1_ref, b1_ref, w2_ref, b2_ref, o_ref):
        p = p_ref[...]
        h = lax.dot_general(
            p, w1_ref[...], (((1,), (1,)), ((), ())),
            preferred_element_type=jnp.float32,
        ) + b1_ref[...]
        h = jnp.maximum(h, 0.0)
        o_ref[...] = lax.dot_general(
            h, w2_ref[...], (((1,), (1,)), ((), ())),
            preferred_element_type=jnp.float32,
        ) + b2_ref[...]

    return pl.pallas_call(
        mk,
        out_shape=jax.ShapeDtypeStruct((BATCH, NUM_CLASSES), jnp.float32),
    )(pooled, W1, b1.reshape(1, DIM), W2, b2.reshape(1, NUM_CLASSES))


def kernel(x, seq_lens, emb, W1, b1, W2, b2):
    del seq_lens
    x2d = x.reshape(BATCH * 2, HALF)
    pooled = _pooled_mean(x2d, emb)
    return _mlp(pooled, W1, b1, W2, b2)
